# baseline (speedup 1.0000x reference)
import math

import jax
import jax.numpy as jnp
from jax.experimental import pallas as pl
from jax.experimental.pallas import tpu as pltpu

D0, H, W = 21, 80, 64
_INV = 1.0 / math.sqrt(1.0 + 1e-3)

_LAYER_PLAN = [
    ('subm', 1, ((1, 1), (1, 1), (1, 1))),
    ('subm', 1, ((1, 1), (1, 1), (1, 1))),
    ('sp',   2, ((1, 1), (1, 1), (1, 1))),
    ('subm', 1, ((1, 1), (1, 1), (1, 1))),
    ('subm', 1, ((1, 1), (1, 1), (1, 1))),
    ('subm', 1, ((1, 1), (1, 1), (1, 1))),
    ('sp',   2, ((0, 0), (1, 1), (1, 1))),
    ('subm', 1, ((1, 1), (1, 1), (1, 1))),
    ('subm', 1, ((1, 1), (1, 1), (1, 1))),
    ('subm', 1, ((1, 1), (1, 1), (1, 1))),
    ('sp',   2, ((0, 0), (0, 0), (0, 0))),
]


def _subm_body(ky, kx, cout):
    def body(x0, x1, x2, w_ref, g_ref, b_ref, m_ref, y_ref):
        acc = jnp.zeros((H, W, cout), jnp.float32)
        for dz, xr in enumerate((x0, x1, x2)):
            for dy in range(ky):
                for dx in range(kx):
                    win = xr[0, dy:dy + H, dx:dx + W, :]
                    acc += jax.lax.dot_general(
                        win, w_ref[dz * ky * kx + dy * kx + dx],
                        (((2,), (0,)), ((), ())),
                        preferred_element_type=jnp.float32)
        y = acc * (_INV * g_ref[0]) + b_ref[0]
        y_ref[0] = jnp.maximum(y, 0.0) * m_ref[0][:, :, None]
    return body


def _sp_body(ky, kx, cout):
    def body(x0, x1, x2, m0, m1, m2, w_ref, g_ref, b_ref, y_ref, mo_ref):
        acc = jnp.zeros((H, W, cout), jnp.float32)
        msum = jnp.zeros((H, W), jnp.float32)
        for dz, (xr, mr) in enumerate(zip((x0, x1, x2), (m0, m1, m2))):
            for dy in range(ky):
                for dx in range(kx):
                    win = xr[0, dy:dy + H, dx:dx + W, :]
                    acc += jax.lax.dot_general(
                        win, w_ref[dz * ky * kx + dy * kx + dx],
                        (((2,), (0,)), ((), ())),
                        preferred_element_type=jnp.float32)
                    msum += mr[0, dy:dy + H, dx:dx + W]
        newm = (msum > 0.0).astype(jnp.float32)
        y = acc * (_INV * g_ref[0]) + b_ref[0]
        y_ref[0] = jnp.maximum(y, 0.0) * newm[:, :, None]
        mo_ref[0] = newm
    return body


def _conv_layer(x, mask, w, gamma, beta, kind, sz, pad):
    kz, ky, kx, cin, cout = w.shape
    pz, py, px = pad
    din = x.shape[0]
    dout = (din + pz[0] + pz[1] - kz) // sz + 1
    hp, wp = H + py[0] + py[1], W + px[0] + px[1]
    xp = jnp.pad(x, ((pz[0], pz[1]), (py[0], py[1]), (px[0], px[1]), (0, 0)))
    wf = w.reshape(kz * ky * kx, cin, cout)
    g2 = gamma.reshape(1, cout)
    b2 = beta.reshape(1, cout)

    def slab(dz):
        return pl.BlockSpec((1, hp, wp, cin), lambda d, dz=dz: (d * sz + dz, 0, 0, 0))

    def mslab(dz):
        return pl.BlockSpec((1, hp, wp), lambda d, dz=dz: (d * sz + dz, 0, 0))

    wspec = pl.BlockSpec((kz * ky * kx, cin, cout), lambda d: (0, 0, 0))
    vspec = pl.BlockSpec((1, cout), lambda d: (0, 0))
    yspec = pl.BlockSpec((1, H, W, cout), lambda d: (d, 0, 0, 0))
    mospec = pl.BlockSpec((1, H, W), lambda d: (d, 0, 0))

    if kind == 'subm':
        y = pl.pallas_call(
            _subm_body(ky, kx, cout),
            grid=(dout,),
            in_specs=[slab(0), slab(1), slab(2), wspec, vspec, vspec, mospec],
            out_specs=yspec,
            out_shape=jax.ShapeDtypeStruct((dout, H, W, cout), jnp.float32),
        )(xp, xp, xp, wf, g2, b2, mask)
        return y, mask
    else:
        mp = jnp.pad(mask, ((pz[0], pz[1]), (py[0], py[1]), (px[0], px[1])))
        y, newm = pl.pallas_call(
            _sp_body(ky, kx, cout),
            grid=(dout,),
            in_specs=[slab(0), slab(1), slab(2), mslab(0), mslab(1), mslab(2),
                      wspec, vspec, vspec],
            out_specs=[yspec, mospec],
            out_shape=[jax.ShapeDtypeStruct((dout, H, W, cout), jnp.float32),
                       jax.ShapeDtypeStruct((dout, H, W), jnp.float32)],
        )(xp, xp, xp, mp, mp, mp, wf, g2, b2)
        return y, newm


def kernel(voxel_features, coors, batch_size, weights, gammas, betas):
    del batch_size
    nvox, cin = voxel_features.shape
    lin = coors[:, 1] * (H * W) + coors[:, 2] * W + coors[:, 3]
    x = jnp.zeros((D0 * H * W, cin), jnp.float32).at[lin].set(voxel_features)
    x = x.reshape(D0, H, W, cin)
    mask = jnp.zeros((D0 * H * W,), jnp.float32).at[lin].set(1.0)
    mask = mask.reshape(D0, H, W)

    for (kind, sz, pad), w, g, b in zip(_LAYER_PLAN, weights, gammas, betas):
        x, mask = _conv_layer(x, mask, w, g, b, kind, sz, pad)

    dd, _, _, c = x.shape
    out = jnp.transpose(x, (3, 0, 1, 2)).reshape(1, c * dd, H, W)
    return out


# speedup vs baseline: 1.0039x; 1.0039x over previous
import math

import jax
import jax.numpy as jnp
from jax.experimental import pallas as pl
from jax.experimental.pallas import tpu as pltpu

D0, H, W = 21, 80, 64
_INV = 1.0 / math.sqrt(1.0 + 1e-3)

_LAYER_PLAN = [
    ('subm', 1, ((1, 1), (1, 1), (1, 1))),
    ('subm', 1, ((1, 1), (1, 1), (1, 1))),
    ('sp',   2, ((1, 1), (1, 1), (1, 1))),
    ('subm', 1, ((1, 1), (1, 1), (1, 1))),
    ('subm', 1, ((1, 1), (1, 1), (1, 1))),
    ('subm', 1, ((1, 1), (1, 1), (1, 1))),
    ('sp',   2, ((0, 0), (1, 1), (1, 1))),
    ('subm', 1, ((1, 1), (1, 1), (1, 1))),
    ('subm', 1, ((1, 1), (1, 1), (1, 1))),
    ('subm', 1, ((1, 1), (1, 1), (1, 1))),
    ('sp',   2, ((0, 0), (0, 0), (0, 0))),
]


def _subm_body(ky, kx, cout):
    def body(x0, x1, x2, w_ref, g_ref, b_ref, m_ref, y_ref):
        acc = jnp.zeros((H, W, cout), jnp.float32)
        for dz, xr in enumerate((x0, x1, x2)):
            xb = xr[0].astype(jnp.bfloat16)
            for dy in range(ky):
                for dx in range(kx):
                    win = xb[dy:dy + H, dx:dx + W, :]
                    acc += jax.lax.dot_general(
                        win, w_ref[dz * ky * kx + dy * kx + dx],
                        (((2,), (0,)), ((), ())),
                        preferred_element_type=jnp.float32)
        y = acc * (_INV * g_ref[0]) + b_ref[0]
        y_ref[0] = jnp.maximum(y, 0.0) * m_ref[0][:, :, None]
    return body


def _sp_body(ky, kx, cout):
    def body(x0, x1, x2, m0, m1, m2, w_ref, g_ref, b_ref, y_ref, mo_ref):
        acc = jnp.zeros((H, W, cout), jnp.float32)
        msum = jnp.zeros((H, W), jnp.float32)
        for dz, (xr, mr) in enumerate(zip((x0, x1, x2), (m0, m1, m2))):
            xb = xr[0].astype(jnp.bfloat16)
            for dy in range(ky):
                for dx in range(kx):
                    win = xb[dy:dy + H, dx:dx + W, :]
                    acc += jax.lax.dot_general(
                        win, w_ref[dz * ky * kx + dy * kx + dx],
                        (((2,), (0,)), ((), ())),
                        preferred_element_type=jnp.float32)
                    msum += mr[0, dy:dy + H, dx:dx + W]
        newm = (msum > 0.0).astype(jnp.float32)
        y = acc * (_INV * g_ref[0]) + b_ref[0]
        y_ref[0] = jnp.maximum(y, 0.0) * newm[:, :, None]
        mo_ref[0] = newm
    return body


def _conv_layer(x, mask, w, gamma, beta, kind, sz, pad):
    kz, ky, kx, cin, cout = w.shape
    pz, py, px = pad
    din = x.shape[0]
    dout = (din + pz[0] + pz[1] - kz) // sz + 1
    hp, wp = H + py[0] + py[1], W + px[0] + px[1]
    xp = jnp.pad(x, ((pz[0], pz[1]), (py[0], py[1]), (px[0], px[1]), (0, 0)))
    wf = w.reshape(kz * ky * kx, cin, cout).astype(jnp.bfloat16)
    g2 = gamma.reshape(1, cout)
    b2 = beta.reshape(1, cout)

    def slab(dz):
        return pl.BlockSpec((1, hp, wp, cin), lambda d, dz=dz: (d * sz + dz, 0, 0, 0))

    def mslab(dz):
        return pl.BlockSpec((1, hp, wp), lambda d, dz=dz: (d * sz + dz, 0, 0))

    wspec = pl.BlockSpec((kz * ky * kx, cin, cout), lambda d: (0, 0, 0))
    vspec = pl.BlockSpec((1, cout), lambda d: (0, 0))
    yspec = pl.BlockSpec((1, H, W, cout), lambda d: (d, 0, 0, 0))
    mospec = pl.BlockSpec((1, H, W), lambda d: (d, 0, 0))

    if kind == 'subm':
        y = pl.pallas_call(
            _subm_body(ky, kx, cout),
            grid=(dout,),
            in_specs=[slab(0), slab(1), slab(2), wspec, vspec, vspec, mospec],
            out_specs=yspec,
            out_shape=jax.ShapeDtypeStruct((dout, H, W, cout), jnp.float32),
        )(xp, xp, xp, wf, g2, b2, mask)
        return y, mask
    else:
        mp = jnp.pad(mask, ((pz[0], pz[1]), (py[0], py[1]), (px[0], px[1])))
        y, newm = pl.pallas_call(
            _sp_body(ky, kx, cout),
            grid=(dout,),
            in_specs=[slab(0), slab(1), slab(2), mslab(0), mslab(1), mslab(2),
                      wspec, vspec, vspec],
            out_specs=[yspec, mospec],
            out_shape=[jax.ShapeDtypeStruct((dout, H, W, cout), jnp.float32),
                       jax.ShapeDtypeStruct((dout, H, W), jnp.float32)],
        )(xp, xp, xp, mp, mp, mp, wf, g2, b2)
        return y, newm


def kernel(voxel_features, coors, batch_size, weights, gammas, betas):
    del batch_size
    nvox, cin = voxel_features.shape
    lin = coors[:, 1] * (H * W) + coors[:, 2] * W + coors[:, 3]
    x = jnp.zeros((D0 * H * W, cin), jnp.float32).at[lin].set(voxel_features)
    x = x.reshape(D0, H, W, cin)
    mask = jnp.zeros((D0 * H * W,), jnp.float32).at[lin].set(1.0)
    mask = mask.reshape(D0, H, W)

    for (kind, sz, pad), w, g, b in zip(_LAYER_PLAN, weights, gammas, betas):
        x, mask = _conv_layer(x, mask, w, g, b, kind, sz, pad)

    dd, _, _, c = x.shape
    out = jnp.transpose(x, (3, 0, 1, 2)).reshape(1, c * dd, H, W)
    return out


# speedup vs baseline: 1.0044x; 1.0006x over previous
import math

import jax
import jax.numpy as jnp
from jax.experimental import pallas as pl
from jax.experimental.pallas import tpu as pltpu

D0, H, W = 21, 80, 64
_INV = 1.0 / math.sqrt(1.0 + 1e-3)

_LAYER_PLAN = [
    ('subm', 1, ((1, 1), (1, 1), (1, 1))),
    ('subm', 1, ((1, 1), (1, 1), (1, 1))),
    ('sp',   2, ((1, 1), (1, 1), (1, 1))),
    ('subm', 1, ((1, 1), (1, 1), (1, 1))),
    ('subm', 1, ((1, 1), (1, 1), (1, 1))),
    ('subm', 1, ((1, 1), (1, 1), (1, 1))),
    ('sp',   2, ((0, 0), (1, 1), (1, 1))),
    ('subm', 1, ((1, 1), (1, 1), (1, 1))),
    ('subm', 1, ((1, 1), (1, 1), (1, 1))),
    ('subm', 1, ((1, 1), (1, 1), (1, 1))),
    ('sp',   2, ((0, 0), (0, 0), (0, 0))),
]


def _subm_body(ky, kx, cout):
    def body(x0, x1, x2, w_ref, g_ref, b_ref, m_ref, y_ref):
        acc = jnp.zeros((H, W, cout), jnp.float32)
        for dz, xr in enumerate((x0, x1, x2)):
            xb = xr[0].astype(jnp.bfloat16)
            for dy in range(ky):
                for dx in range(kx):
                    win = xb[dy:dy + H, dx:dx + W, :]
                    acc += jax.lax.dot_general(
                        win, w_ref[dz * ky * kx + dy * kx + dx],
                        (((2,), (0,)), ((), ())),
                        preferred_element_type=jnp.float32)
        y = acc * (_INV * g_ref[0]) + b_ref[0]
        y_ref[0] = jnp.maximum(y, 0.0) * m_ref[0][:, :, None]
    return body


def _sp_body(ky, kx, cout):
    def body(x0, x1, x2, m0, m1, m2, w_ref, g_ref, b_ref, y_ref, mo_ref):
        acc = jnp.zeros((H, W, cout), jnp.float32)
        msum = jnp.zeros((H, W), jnp.float32)
        for dz, (xr, mr) in enumerate(zip((x0, x1, x2), (m0, m1, m2))):
            xb = xr[0].astype(jnp.bfloat16)
            for dy in range(ky):
                for dx in range(kx):
                    win = xb[dy:dy + H, dx:dx + W, :]
                    acc += jax.lax.dot_general(
                        win, w_ref[dz * ky * kx + dy * kx + dx],
                        (((2,), (0,)), ((), ())),
                        preferred_element_type=jnp.float32)
                    msum += mr[0, dy:dy + H, dx:dx + W]
        newm = (msum > 0.0).astype(jnp.float32)
        y = acc * (_INV * g_ref[0]) + b_ref[0]
        y_ref[0] = jnp.maximum(y, 0.0) * newm[:, :, None]
        mo_ref[0] = newm
    return body


def _conv_layer(x, mask, w, gamma, beta, kind, sz, pad):
    kz, ky, kx, cin, cout = w.shape
    pz, py, px = pad
    din = x.shape[0]
    dout = (din + pz[0] + pz[1] - kz) // sz + 1
    hp, wp = H + py[0] + py[1], W + px[0] + px[1]
    xp = jnp.pad(x, ((pz[0], pz[1]), (py[0], py[1]), (px[0], px[1]), (0, 0)))
    wf = w.reshape(kz * ky * kx, cin, cout).astype(jnp.bfloat16)
    g2 = gamma.reshape(1, cout)
    b2 = beta.reshape(1, cout)

    def slab(dz):
        return pl.BlockSpec((1, hp, wp, cin), lambda d, dz=dz: (d * sz + dz, 0, 0, 0))

    def mslab(dz):
        return pl.BlockSpec((1, hp, wp), lambda d, dz=dz: (d * sz + dz, 0, 0))

    wspec = pl.BlockSpec((kz * ky * kx, cin, cout), lambda d: (0, 0, 0))
    vspec = pl.BlockSpec((1, cout), lambda d: (0, 0))
    yspec = pl.BlockSpec((1, H, W, cout), lambda d: (d, 0, 0, 0))
    mospec = pl.BlockSpec((1, H, W), lambda d: (d, 0, 0))

    cparams = pltpu.CompilerParams(dimension_semantics=("parallel",))
    if kind == 'subm':
        y = pl.pallas_call(
            _subm_body(ky, kx, cout),
            grid=(dout,),
            in_specs=[slab(0), slab(1), slab(2), wspec, vspec, vspec, mospec],
            out_specs=yspec,
            out_shape=jax.ShapeDtypeStruct((dout, H, W, cout), jnp.float32),
            compiler_params=cparams,
        )(xp, xp, xp, wf, g2, b2, mask)
        return y, mask
    else:
        mp = jnp.pad(mask, ((pz[0], pz[1]), (py[0], py[1]), (px[0], px[1])))
        y, newm = pl.pallas_call(
            _sp_body(ky, kx, cout),
            grid=(dout,),
            in_specs=[slab(0), slab(1), slab(2), mslab(0), mslab(1), mslab(2),
                      wspec, vspec, vspec],
            out_specs=[yspec, mospec],
            out_shape=[jax.ShapeDtypeStruct((dout, H, W, cout), jnp.float32),
                       jax.ShapeDtypeStruct((dout, H, W), jnp.float32)],
            compiler_params=cparams,
        )(xp, xp, xp, mp, mp, mp, wf, g2, b2)
        return y, newm


def kernel(voxel_features, coors, batch_size, weights, gammas, betas):
    del batch_size
    nvox, cin = voxel_features.shape
    lin = coors[:, 1] * (H * W) + coors[:, 2] * W + coors[:, 3]
    x = jnp.zeros((D0 * H * W, cin), jnp.float32).at[lin].set(voxel_features)
    x = x.reshape(D0, H, W, cin)
    mask = jnp.zeros((D0 * H * W,), jnp.float32).at[lin].set(1.0)
    mask = mask.reshape(D0, H, W)

    for (kind, sz, pad), w, g, b in zip(_LAYER_PLAN, weights, gammas, betas):
        x, mask = _conv_layer(x, mask, w, g, b, kind, sz, pad)

    dd, _, _, c = x.shape
    out = jnp.transpose(x, (3, 0, 1, 2)).reshape(1, c * dd, H, W)
    return out


# speedup vs baseline: 1.3296x; 1.3237x over previous
import math

import jax
import jax.numpy as jnp
from jax.experimental import pallas as pl
from jax.experimental.pallas import tpu as pltpu

D0, H, W = 21, 80, 64
_INV = 1.0 / math.sqrt(1.0 + 1e-3)

_LAYER_PLAN = [
    ('subm', 1, ((1, 1), (1, 1), (1, 1))),
    ('subm', 1, ((1, 1), (1, 1), (1, 1))),
    ('sp',   2, ((1, 1), (1, 1), (1, 1))),
    ('subm', 1, ((1, 1), (1, 1), (1, 1))),
    ('subm', 1, ((1, 1), (1, 1), (1, 1))),
    ('subm', 1, ((1, 1), (1, 1), (1, 1))),
    ('sp',   2, ((0, 0), (1, 1), (1, 1))),
    ('subm', 1, ((1, 1), (1, 1), (1, 1))),
    ('subm', 1, ((1, 1), (1, 1), (1, 1))),
    ('subm', 1, ((1, 1), (1, 1), (1, 1))),
    ('sp',   2, ((0, 0), (0, 0), (0, 0))),
]


def _im2col_dot(xr, w_ref, dz, ky, kx):
    xb = xr[0].astype(jnp.bfloat16)
    wins = [xb[dy:dy + H, dx:dx + W, :]
            for dy in range(ky) for dx in range(kx)]
    cat = wins[0] if len(wins) == 1 else jnp.concatenate(wins, axis=-1)
    return jax.lax.dot_general(
        cat, w_ref[dz], (((2,), (0,)), ((), ())),
        preferred_element_type=jnp.float32)


def _subm_body(ky, kx, cout):
    def body(x0, x1, x2, w_ref, g_ref, b_ref, m_ref, y_ref):
        acc = jnp.zeros((H, W, cout), jnp.float32)
        for dz, xr in enumerate((x0, x1, x2)):
            acc += _im2col_dot(xr, w_ref, dz, ky, kx)
        y = acc * (_INV * g_ref[0]) + b_ref[0]
        y_ref[0] = jnp.maximum(y, 0.0) * m_ref[0][:, :, None]
    return body


def _sp_body(ky, kx, cout):
    def body(x0, x1, x2, m0, m1, m2, w_ref, g_ref, b_ref, y_ref, mo_ref):
        acc = jnp.zeros((H, W, cout), jnp.float32)
        msum = jnp.zeros((H, W), jnp.float32)
        for dz, (xr, mr) in enumerate(zip((x0, x1, x2), (m0, m1, m2))):
            acc += _im2col_dot(xr, w_ref, dz, ky, kx)
            for dy in range(ky):
                for dx in range(kx):
                    msum += mr[0, dy:dy + H, dx:dx + W]
        newm = (msum > 0.0).astype(jnp.float32)
        y = acc * (_INV * g_ref[0]) + b_ref[0]
        y_ref[0] = jnp.maximum(y, 0.0) * newm[:, :, None]
        mo_ref[0] = newm
    return body


def _conv_layer(x, mask, w, gamma, beta, kind, sz, pad):
    kz, ky, kx, cin, cout = w.shape
    pz, py, px = pad
    din = x.shape[0]
    dout = (din + pz[0] + pz[1] - kz) // sz + 1
    hp, wp = H + py[0] + py[1], W + px[0] + px[1]
    xp = jnp.pad(x, ((pz[0], pz[1]), (py[0], py[1]), (px[0], px[1]), (0, 0)))
    wf = w.reshape(kz, ky * kx * cin, cout).astype(jnp.bfloat16)
    g2 = gamma.reshape(1, cout)
    b2 = beta.reshape(1, cout)

    def slab(dz):
        return pl.BlockSpec((1, hp, wp, cin), lambda d, dz=dz: (d * sz + dz, 0, 0, 0))

    def mslab(dz):
        return pl.BlockSpec((1, hp, wp), lambda d, dz=dz: (d * sz + dz, 0, 0))

    wspec = pl.BlockSpec((kz, ky * kx * cin, cout), lambda d: (0, 0, 0))
    vspec = pl.BlockSpec((1, cout), lambda d: (0, 0))
    yspec = pl.BlockSpec((1, H, W, cout), lambda d: (d, 0, 0, 0))
    mospec = pl.BlockSpec((1, H, W), lambda d: (d, 0, 0))

    cparams = pltpu.CompilerParams(dimension_semantics=("parallel",))
    if kind == 'subm':
        y = pl.pallas_call(
            _subm_body(ky, kx, cout),
            grid=(dout,),
            in_specs=[slab(0), slab(1), slab(2), wspec, vspec, vspec, mospec],
            out_specs=yspec,
            out_shape=jax.ShapeDtypeStruct((dout, H, W, cout), jnp.float32),
            compiler_params=cparams,
        )(xp, xp, xp, wf, g2, b2, mask)
        return y, mask
    else:
        mp = jnp.pad(mask, ((pz[0], pz[1]), (py[0], py[1]), (px[0], px[1])))
        y, newm = pl.pallas_call(
            _sp_body(ky, kx, cout),
            grid=(dout,),
            in_specs=[slab(0), slab(1), slab(2), mslab(0), mslab(1), mslab(2),
                      wspec, vspec, vspec],
            out_specs=[yspec, mospec],
            out_shape=[jax.ShapeDtypeStruct((dout, H, W, cout), jnp.float32),
                       jax.ShapeDtypeStruct((dout, H, W), jnp.float32)],
            compiler_params=cparams,
        )(xp, xp, xp, mp, mp, mp, wf, g2, b2)
        return y, newm


def kernel(voxel_features, coors, batch_size, weights, gammas, betas):
    del batch_size
    nvox, cin = voxel_features.shape
    lin = coors[:, 1] * (H * W) + coors[:, 2] * W + coors[:, 3]
    x = jnp.zeros((D0 * H * W, cin), jnp.float32).at[lin].set(voxel_features)
    x = x.reshape(D0, H, W, cin)
    mask = jnp.zeros((D0 * H * W,), jnp.float32).at[lin].set(1.0)
    mask = mask.reshape(D0, H, W)

    for (kind, sz, pad), w, g, b in zip(_LAYER_PLAN, weights, gammas, betas):
        x, mask = _conv_layer(x, mask, w, g, b, kind, sz, pad)

    dd, _, _, c = x.shape
    out = jnp.transpose(x, (3, 0, 1, 2)).reshape(1, c * dd, H, W)
    return out


# speedup vs baseline: 1.4190x; 1.0672x over previous
import math

import jax
import jax.numpy as jnp
from jax.experimental import pallas as pl
from jax.experimental.pallas import tpu as pltpu

D0, H, W = 21, 80, 64
_INV = 1.0 / math.sqrt(1.0 + 1e-3)

_LAYER_PLAN = [
    ('subm', 1, ((1, 1), (1, 1), (1, 1))),
    ('subm', 1, ((1, 1), (1, 1), (1, 1))),
    ('sp',   2, ((1, 1), (1, 1), (1, 1))),
    ('subm', 1, ((1, 1), (1, 1), (1, 1))),
    ('subm', 1, ((1, 1), (1, 1), (1, 1))),
    ('subm', 1, ((1, 1), (1, 1), (1, 1))),
    ('sp',   2, ((0, 0), (1, 1), (1, 1))),
    ('subm', 1, ((1, 1), (1, 1), (1, 1))),
    ('subm', 1, ((1, 1), (1, 1), (1, 1))),
    ('subm', 1, ((1, 1), (1, 1), (1, 1))),
    ('sp',   2, ((0, 0), (0, 0), (0, 0))),
]


def _im2col_dot(xr, w_ref, dz, ky, kx):
    xb = xr[0]
    wins = [xb[dy:dy + H, dx:dx + W, :]
            for dy in range(ky) for dx in range(kx)]
    cat = wins[0] if len(wins) == 1 else jnp.concatenate(wins, axis=-1)
    return jax.lax.dot_general(
        cat, w_ref[dz], (((2,), (0,)), ((), ())),
        preferred_element_type=jnp.float32)


def _subm_body(ky, kx, cout):
    def body(x0, x1, x2, w_ref, g_ref, b_ref, m_ref, y_ref):
        acc = jnp.zeros((H, W, cout), jnp.float32)
        for dz, xr in enumerate((x0, x1, x2)):
            acc += _im2col_dot(xr, w_ref, dz, ky, kx)
        y = acc * (_INV * g_ref[0]) + b_ref[0]
        y = jnp.maximum(y, 0.0) * m_ref[0][:, :, None].astype(jnp.float32)
        y_ref[0] = y.astype(jnp.bfloat16)
    return body


def _sp_body(ky, kx, cout):
    def body(x0, x1, x2, m0, m1, m2, w_ref, g_ref, b_ref, y_ref, mo_ref):
        acc = jnp.zeros((H, W, cout), jnp.float32)
        msum = jnp.zeros((H, W), jnp.float32)
        for dz, (xr, mr) in enumerate(zip((x0, x1, x2), (m0, m1, m2))):
            acc += _im2col_dot(xr, w_ref, dz, ky, kx)
            for dy in range(ky):
                for dx in range(kx):
                    msum += mr[0, dy:dy + H, dx:dx + W].astype(jnp.float32)
        newm = (msum > 0.0).astype(jnp.float32)
        y = acc * (_INV * g_ref[0]) + b_ref[0]
        y = jnp.maximum(y, 0.0) * newm[:, :, None]
        y_ref[0] = y.astype(jnp.bfloat16)
        mo_ref[0] = newm.astype(jnp.bfloat16)
    return body


def _conv_layer(x, mask, w, gamma, beta, kind, sz, pad):
    kz, ky, kx, cin, cout = w.shape
    pz, py, px = pad
    din = x.shape[0]
    dout = (din + pz[0] + pz[1] - kz) // sz + 1
    hp, wp = H + py[0] + py[1], W + px[0] + px[1]
    xp = jnp.pad(x, ((pz[0], pz[1]), (py[0], py[1]), (px[0], px[1]), (0, 0)))
    wf = w.reshape(kz, ky * kx * cin, cout).astype(jnp.bfloat16)
    g2 = gamma.reshape(1, cout)
    b2 = beta.reshape(1, cout)

    def slab(dz):
        return pl.BlockSpec((1, hp, wp, cin), lambda d, dz=dz: (d * sz + dz, 0, 0, 0))

    def mslab(dz):
        return pl.BlockSpec((1, hp, wp), lambda d, dz=dz: (d * sz + dz, 0, 0))

    wspec = pl.BlockSpec((kz, ky * kx * cin, cout), lambda d: (0, 0, 0))
    vspec = pl.BlockSpec((1, cout), lambda d: (0, 0))
    yspec = pl.BlockSpec((1, H, W, cout), lambda d: (d, 0, 0, 0))
    mospec = pl.BlockSpec((1, H, W), lambda d: (d, 0, 0))

    cparams = pltpu.CompilerParams(dimension_semantics=("parallel",))
    if kind == 'subm':
        y = pl.pallas_call(
            _subm_body(ky, kx, cout),
            grid=(dout,),
            in_specs=[slab(0), slab(1), slab(2), wspec, vspec, vspec, mospec],
            out_specs=yspec,
            out_shape=jax.ShapeDtypeStruct((dout, H, W, cout), jnp.bfloat16),
            compiler_params=cparams,
        )(xp, xp, xp, wf, g2, b2, mask)
        return y, mask
    else:
        mp = jnp.pad(mask, ((pz[0], pz[1]), (py[0], py[1]), (px[0], px[1])))
        y, newm = pl.pallas_call(
            _sp_body(ky, kx, cout),
            grid=(dout,),
            in_specs=[slab(0), slab(1), slab(2), mslab(0), mslab(1), mslab(2),
                      wspec, vspec, vspec],
            out_specs=[yspec, mospec],
            out_shape=[jax.ShapeDtypeStruct((dout, H, W, cout), jnp.bfloat16),
                       jax.ShapeDtypeStruct((dout, H, W), jnp.bfloat16)],
            compiler_params=cparams,
        )(xp, xp, xp, mp, mp, mp, wf, g2, b2)
        return y, newm


def kernel(voxel_features, coors, batch_size, weights, gammas, betas):
    del batch_size
    nvox, cin = voxel_features.shape
    lin = coors[:, 1] * (H * W) + coors[:, 2] * W + coors[:, 3]
    x = jnp.zeros((D0 * H * W, cin), jnp.bfloat16).at[lin].set(
        voxel_features.astype(jnp.bfloat16))
    x = x.reshape(D0, H, W, cin)
    mask = jnp.zeros((D0 * H * W,), jnp.bfloat16).at[lin].set(1.0)
    mask = mask.reshape(D0, H, W)

    for (kind, sz, pad), w, g, b in zip(_LAYER_PLAN, weights, gammas, betas):
        x, mask = _conv_layer(x, mask, w, g, b, kind, sz, pad)

    dd, _, _, c = x.shape
    out = jnp.transpose(x.astype(jnp.float32), (3, 0, 1, 2)).reshape(1, c * dd, H, W)
    return out
